# baseline (speedup 1.0000x reference)
import functools

import jax
import jax.numpy as jnp
from jax import lax
from jax.experimental import pallas as pl
from jax.experimental.pallas import tpu as pltpu
from jax.experimental.pallas import tpu_sc as plsc

_B = 4
_S = 2048
_D = 128
_L = 16
_UNROLL = 4

_info = plsc.get_sparse_core_info()
_NC = _info.num_cores
_NS = _info.num_subcores
_NW = _NC * _NS
_SPW = _S // _NW


def _emb_body(x_hbm, tok_hbm, pos_hbm, out_hbm, idx_v, tok_v, pos_v,
              sem_i, sem_p, sem_g, sem_o):
    wid = lax.axis_index("s") * _NC + lax.axis_index("c")
    s0 = wid * _SPW

    cps_idx = [
        pltpu.async_copy(x_hbm.at[b, pl.ds(s0, _SPW)], idx_v.at[b], sem_i)
        for b in range(_B)
    ]
    cp_pos = pltpu.async_copy(pos_hbm.at[pl.ds(s0, _SPW)], pos_v, sem_p)
    for cp in cps_idx:
        cp.wait()
    cps_g = [
        pltpu.async_copy(tok_hbm.at[idx_v.at[b]], tok_v.at[b], sem_g)
        for b in range(_B)
    ]
    cp_pos.wait()

    def _rows(b):
        def body(r, carry):
            for p in range(_UNROLL):
                rr = _UNROLL * r + p
                for j in range(_D // _L):
                    sl = pl.ds(j * _L, _L)
                    plsc.addupdate(tok_v.at[b, rr, sl], pos_v[rr, sl])
            return carry
        return body

    cps_o = []
    for b in range(_B):
        cps_g[b].wait()
        lax.fori_loop(0, _SPW // _UNROLL, _rows(b), 0)
        cps_o.append(
            pltpu.async_copy(tok_v.at[b], out_hbm.at[b, pl.ds(s0, _SPW)],
                             sem_o)
        )
    for cp in cps_o:
        cp.wait()


_emb = functools.partial(
    pl.kernel,
    out_type=jax.ShapeDtypeStruct((_B, _S, _D), jnp.float32),
    mesh=plsc.VectorSubcoreMesh(core_axis_name="c", subcore_axis_name="s"),
    scratch_types=[
        pltpu.VMEM((_B, _SPW), jnp.int32),
        pltpu.VMEM((_B, _SPW, _D), jnp.float32),
        pltpu.VMEM((_SPW, _D), jnp.float32),
        pltpu.SemaphoreType.DMA,
        pltpu.SemaphoreType.DMA,
        pltpu.SemaphoreType.DMA,
        pltpu.SemaphoreType.DMA,
    ],
)(_emb_body)


@jax.jit
def kernel(x, tok_emb_weight, pos_emb_weight):
    return _emb(x.astype(jnp.int32), tok_emb_weight, pos_emb_weight)


# speedup vs baseline: 1.0260x; 1.0260x over previous
import functools

import jax
import jax.numpy as jnp
from jax import lax
from jax.experimental import pallas as pl
from jax.experimental.pallas import tpu as pltpu
from jax.experimental.pallas import tpu_sc as plsc

_B = 4
_S = 2048
_D = 128
_L = 16
_UNROLL = 2

_info = plsc.get_sparse_core_info()
_NC = _info.num_cores
_NS = _info.num_subcores
_NW = _NC * _NS
_SPW = _S // _NW


def _emb_body(x_hbm, tok_hbm, pos_hbm, out_hbm, idx_v, tok_v, pos_v,
              sem_i, sem_p, sem_g, sem_o):
    wid = lax.axis_index("s") * _NC + lax.axis_index("c")
    s0 = wid * _SPW

    cps_idx = [
        pltpu.async_copy(x_hbm.at[b, pl.ds(s0, _SPW)], idx_v.at[b], sem_i)
        for b in range(_B)
    ]
    cp_pos = pltpu.async_copy(pos_hbm.at[pl.ds(s0, _SPW)], pos_v, sem_p)
    for cp in cps_idx:
        cp.wait()
    cps_g = [
        pltpu.async_copy(tok_hbm.at[idx_v.at[b]], tok_v.at[b], sem_g)
        for b in range(_B)
    ]
    cp_pos.wait()

    def _rows(b):
        def body(r, carry):
            for p in range(_UNROLL):
                rr = _UNROLL * r + p
                for j in range(_D // _L):
                    sl = pl.ds(j * _L, _L)
                    plsc.addupdate(tok_v.at[b, rr, sl], pos_v[rr, sl])
            return carry
        return body

    cps_o = []
    for b in range(_B):
        cps_g[b].wait()
        lax.fori_loop(0, _SPW // _UNROLL, _rows(b), 0)
        cps_o.append(
            pltpu.async_copy(tok_v.at[b], out_hbm.at[b, pl.ds(s0, _SPW)],
                             sem_o)
        )
    for cp in cps_o:
        cp.wait()


_emb = functools.partial(
    pl.kernel,
    out_type=jax.ShapeDtypeStruct((_B, _S, _D), jnp.float32),
    mesh=plsc.VectorSubcoreMesh(core_axis_name="c", subcore_axis_name="s"),
    scratch_types=[
        pltpu.VMEM((_B, _SPW), jnp.int32),
        pltpu.VMEM((_B, _SPW, _D), jnp.float32),
        pltpu.VMEM((_SPW, _D), jnp.float32),
        pltpu.SemaphoreType.DMA,
        pltpu.SemaphoreType.DMA,
        pltpu.SemaphoreType.DMA,
        pltpu.SemaphoreType.DMA,
    ],
)(_emb_body)


@jax.jit
def kernel(x, tok_emb_weight, pos_emb_weight):
    return _emb(x.astype(jnp.int32), tok_emb_weight, pos_emb_weight)


# speedup vs baseline: 1.0271x; 1.0010x over previous
import functools

import jax
import jax.numpy as jnp
from jax import lax
from jax.experimental import pallas as pl
from jax.experimental.pallas import tpu as pltpu
from jax.experimental.pallas import tpu_sc as plsc

_B = 4
_S = 2048
_D = 128
_L = 16

_info = plsc.get_sparse_core_info()
_NC = _info.num_cores
_NS = _info.num_subcores
_NW = _NC * _NS
_SPW = _S // _NW
_ROWS = _B * _SPW
_GSZ = 128


def _emb_body(x_hbm, tok_hbm, pos_hbm, out_hbm, idx_v, tok_v, pos_v,
              sem_i, sem_p, sem_g, sem_o):
    wid = lax.axis_index("s") * _NC + lax.axis_index("c")
    s0 = wid * _SPW

    cps_idx = [
        pltpu.async_copy(x_hbm.at[b, pl.ds(s0, _SPW)],
                         idx_v.at[pl.ds(b * _SPW, _SPW)], sem_i)
        for b in range(_B)
    ]
    cp_pos = pltpu.async_copy(pos_hbm.at[pl.ds(s0, _SPW)], pos_v, sem_p)
    for cp in cps_idx:
        cp.wait()
    cps_g = [
        pltpu.async_copy(
            tok_hbm.at[idx_v.at[pl.ds(k * _GSZ, _GSZ)]],
            tok_v.at[pl.ds(k * _GSZ, _GSZ)],
            sem_g,
        )
        for k in range(_ROWS // _GSZ)
    ]
    cp_pos.wait()

    def _rows(b):
        def body(r, carry):
            for j in range(_D // _L):
                sl = pl.ds(j * _L, _L)
                plsc.addupdate(tok_v.at[b * _SPW + r, sl], pos_v[r, sl])
            return carry
        return body

    cps_o = []
    for k in range(_ROWS // _GSZ):
        cps_g[k].wait()
        for b in (2 * k, 2 * k + 1):
            lax.fori_loop(0, _SPW, _rows(b), 0)
            cps_o.append(
                pltpu.async_copy(
                    tok_v.at[pl.ds(b * _SPW, _SPW)],
                    out_hbm.at[b, pl.ds(s0, _SPW)],
                    sem_o,
                )
            )
    for cp in cps_o:
        cp.wait()


_emb = functools.partial(
    pl.kernel,
    out_type=jax.ShapeDtypeStruct((_B, _S, _D), jnp.float32),
    mesh=plsc.VectorSubcoreMesh(core_axis_name="c", subcore_axis_name="s"),
    scratch_types=[
        pltpu.VMEM((_ROWS,), jnp.int32),
        pltpu.VMEM((_ROWS, _D), jnp.float32),
        pltpu.VMEM((_SPW, _D), jnp.float32),
        pltpu.SemaphoreType.DMA,
        pltpu.SemaphoreType.DMA,
        pltpu.SemaphoreType.DMA,
        pltpu.SemaphoreType.DMA,
    ],
)(_emb_body)


@jax.jit
def kernel(x, tok_emb_weight, pos_emb_weight):
    return _emb(x.astype(jnp.int32), tok_emb_weight, pos_emb_weight)


# speedup vs baseline: 1.0411x; 1.0137x over previous
import functools

import jax
import jax.numpy as jnp
from jax import lax
from jax.experimental import pallas as pl
from jax.experimental.pallas import tpu as pltpu
from jax.experimental.pallas import tpu_sc as plsc

_B = 4
_S = 2048
_D = 128
_L = 16

_info = plsc.get_sparse_core_info()
_NC = _info.num_cores
_NS = _info.num_subcores
_NW = _NC * _NS
_SPW = _S // _NW


def _emb_body(x_hbm, tok_hbm, pos_hbm, out_hbm, idx_v, tok_v, pos_v,
              sem_i, sem_p, sem_g, sem_o):
    wid = lax.axis_index("s") * _NC + lax.axis_index("c")
    s0 = wid * _SPW

    cps_idx = [
        pltpu.async_copy(x_hbm.at[b, pl.ds(s0, _SPW)], idx_v.at[b], sem_i)
        for b in range(_B)
    ]
    cp_pos = pltpu.async_copy(pos_hbm.at[pl.ds(s0, _SPW)], pos_v, sem_p)
    for cp in cps_idx:
        cp.wait()
    cps_g = [
        pltpu.async_copy(tok_hbm.at[idx_v.at[b]], tok_v.at[b], sem_g)
        for b in range(_B)
    ]
    cp_pos.wait()

    def _row(b):
        def body(r, carry):
            for j in range(_D // _L):
                sl = pl.ds(j * _L, _L)
                plsc.addupdate(tok_v.at[b, r, sl], pos_v[r, sl])
            return carry
        return body

    cps_o = []
    for b in range(_B):
        cps_g[b].wait()
        lax.fori_loop(0, _SPW, _row(b), 0)
        cps_o.append(
            pltpu.async_copy(tok_v.at[b], out_hbm.at[b, pl.ds(s0, _SPW)],
                             sem_o)
        )
    for cp in cps_o:
        cp.wait()


_emb = functools.partial(
    pl.kernel,
    out_type=jax.ShapeDtypeStruct((_B, _S, _D), jnp.float32),
    mesh=plsc.VectorSubcoreMesh(core_axis_name="c", subcore_axis_name="s"),
    scratch_types=[
        pltpu.VMEM((_B, _SPW), jnp.int32),
        pltpu.VMEM((_B, _SPW, _D), jnp.float32),
        pltpu.VMEM((_SPW, _D), jnp.float32),
        pltpu.SemaphoreType.DMA,
        pltpu.SemaphoreType.DMA,
        pltpu.SemaphoreType.DMA,
        pltpu.SemaphoreType.DMA,
    ],
)(_emb_body)


@jax.jit
def kernel(x, tok_emb_weight, pos_emb_weight):
    return _emb(x.astype(jnp.int32), tok_emb_weight, pos_emb_weight)


# speedup vs baseline: 1.0415x; 1.0004x over previous
import functools

import jax
import jax.numpy as jnp
from jax import lax
from jax.experimental import pallas as pl
from jax.experimental.pallas import tpu as pltpu
from jax.experimental.pallas import tpu_sc as plsc

_B = 4
_S = 2048
_D = 128
_L = 16

_info = plsc.get_sparse_core_info()
_NC = _info.num_cores
_NS = _info.num_subcores
_NW = _NC * _NS
_SPW = _S // _NW


def _emb_body(x_hbm, tok_hbm, pos_hbm, out_hbm, idx_v, tok_v, pos_v,
              sem_i, sem_p, sem_g, sem_o):
    wid = lax.axis_index("s") * _NC + lax.axis_index("c")
    s0 = wid * _SPW

    cps_idx = [
        pltpu.async_copy(x_hbm.at[b, pl.ds(s0, _SPW)], idx_v.at[b], sem_i)
        for b in range(_B)
    ]
    cp_pos = pltpu.async_copy(pos_hbm.at[pl.ds(s0, _SPW)], pos_v, sem_p)
    cps_g = []
    for b in range(_B):
        cps_idx[b].wait()
        cps_g.append(
            pltpu.async_copy(tok_hbm.at[idx_v.at[b]], tok_v.at[b], sem_g)
        )
    cp_pos.wait()

    def _row(b):
        def body(r, carry):
            for j in range(_D // _L):
                sl = pl.ds(j * _L, _L)
                plsc.addupdate(tok_v.at[b, r, sl], pos_v[r, sl])
            return carry
        return body

    cps_o = []
    for b in range(_B):
        cps_g[b].wait()
        lax.fori_loop(0, _SPW, _row(b), 0)
        cps_o.append(
            pltpu.async_copy(tok_v.at[b], out_hbm.at[b, pl.ds(s0, _SPW)],
                             sem_o)
        )
    for cp in cps_o:
        cp.wait()


_emb = functools.partial(
    pl.kernel,
    out_type=jax.ShapeDtypeStruct((_B, _S, _D), jnp.float32),
    mesh=plsc.VectorSubcoreMesh(core_axis_name="c", subcore_axis_name="s"),
    scratch_types=[
        pltpu.VMEM((_B, _SPW), jnp.int32),
        pltpu.VMEM((_B, _SPW, _D), jnp.float32),
        pltpu.VMEM((_SPW, _D), jnp.float32),
        pltpu.SemaphoreType.DMA,
        pltpu.SemaphoreType.DMA,
        pltpu.SemaphoreType.DMA,
        pltpu.SemaphoreType.DMA,
    ],
)(_emb_body)


@jax.jit
def kernel(x, tok_emb_weight, pos_emb_weight):
    return _emb(x.astype(jnp.int32), tok_emb_weight, pos_emb_weight)


# speedup vs baseline: 1.0483x; 1.0065x over previous
import functools

import jax
import jax.numpy as jnp
from jax import lax
from jax.experimental import pallas as pl
from jax.experimental.pallas import tpu as pltpu
from jax.experimental.pallas import tpu_sc as plsc

_B = 4
_S = 2048
_D = 128
_L = 16

_info = plsc.get_sparse_core_info()
_NC = _info.num_cores
_NS = _info.num_subcores
_NW = _NC * _NS
_SPW = _S // _NW


def _emb_body(x_hbm, tok_hbm, pos_hbm, out_hbm, idx_v, tok_v, pos_v,
              sem_i, sem_p, sem_g, sem_o):
    wid = lax.axis_index("s") * _NC + lax.axis_index("c")
    s0 = wid * _SPW

    cps_idx = [
        pltpu.async_copy(x_hbm.at[b, pl.ds(s0, _SPW)], idx_v.at[b], sem_i)
        for b in range(_B)
    ]
    cp_pos = pltpu.async_copy(pos_hbm.at[pl.ds(s0, _SPW)], pos_v, sem_p)
    cps_g = []
    for b in range(_B):
        cps_idx[b].wait()
        cps_g.append(
            pltpu.async_copy(tok_hbm.at[idx_v.at[b]], tok_v.at[b], sem_g)
        )
    cp_pos.wait()

    def _row(b, r0):
        def body(r, carry):
            for j in range(_D // _L):
                sl = pl.ds(j * _L, _L)
                plsc.addupdate(tok_v.at[b, r0 + r, sl], pos_v[r0 + r, sl])
            return carry
        return body

    cps_o = []
    for b in range(_B):
        cps_g[b].wait()
        if b < _B - 1:
            lax.fori_loop(0, _SPW, _row(b, 0), 0)
            cps_o.append(
                pltpu.async_copy(tok_v.at[b], out_hbm.at[b, pl.ds(s0, _SPW)],
                                 sem_o)
            )
        else:
            half = _SPW // 2
            for h in range(2):
                lax.fori_loop(0, half, _row(b, h * half), 0)
                cps_o.append(
                    pltpu.async_copy(
                        tok_v.at[b, pl.ds(h * half, half)],
                        out_hbm.at[b, pl.ds(s0 + h * half, half)],
                        sem_o,
                    )
                )
    for cp in cps_o:
        cp.wait()


_emb = functools.partial(
    pl.kernel,
    out_type=jax.ShapeDtypeStruct((_B, _S, _D), jnp.float32),
    mesh=plsc.VectorSubcoreMesh(core_axis_name="c", subcore_axis_name="s"),
    scratch_types=[
        pltpu.VMEM((_B, _SPW), jnp.int32),
        pltpu.VMEM((_B, _SPW, _D), jnp.float32),
        pltpu.VMEM((_SPW, _D), jnp.float32),
        pltpu.SemaphoreType.DMA,
        pltpu.SemaphoreType.DMA,
        pltpu.SemaphoreType.DMA,
        pltpu.SemaphoreType.DMA,
    ],
)(_emb_body)


@jax.jit
def kernel(x, tok_emb_weight, pos_emb_weight):
    return _emb(x.astype(jnp.int32), tok_emb_weight, pos_emb_weight)
